# baseline (speedup 1.0000x reference)
import jax
import jax.numpy as jnp
from jax.experimental import pallas as pl
from jax.experimental.pallas import tpu as pltpu

_NUM_BUFS = 4
_CHUNKS_PER_BUF = 8
_NUM_CHUNKS = _NUM_BUFS * _CHUNKS_PER_BUF


def _copy_kernel(in_hbm, out_hbm, b0, b1, b2, b3, in_sems, out_sems):
    rows, _ = in_hbm.shape
    chunk = rows // _NUM_CHUNKS
    bufs = (b0, b1, b2, b3)

    def slot(i):
        return bufs[i % _NUM_BUFS].at[i // _NUM_BUFS]

    def copy_in(i):
        return pltpu.make_async_copy(
            in_hbm.at[pl.ds(i * chunk, chunk)], slot(i), in_sems.at[i])

    def copy_out(i):
        return pltpu.make_async_copy(
            slot(i), out_hbm.at[pl.ds(i * chunk, chunk)], out_sems.at[i])

    for i in range(_NUM_CHUNKS):
        copy_in(i).start()
    for i in range(_NUM_CHUNKS):
        copy_in(i).wait()
        copy_out(i).start()
    for i in range(_NUM_CHUNKS):
        copy_out(i).wait()


def kernel(inputs, embedding_table):
    del embedding_table
    rows, cols = inputs.shape
    chunk = rows // _NUM_CHUNKS
    buf_t = pltpu.VMEM((_CHUNKS_PER_BUF, chunk, cols), inputs.dtype)
    return pl.pallas_call(
        _copy_kernel,
        out_shape=jax.ShapeDtypeStruct(inputs.shape, inputs.dtype),
        in_specs=[pl.BlockSpec(memory_space=pl.ANY)],
        out_specs=pl.BlockSpec(memory_space=pl.ANY),
        scratch_shapes=[
            buf_t, buf_t, buf_t, buf_t,
            pltpu.SemaphoreType.DMA((_NUM_CHUNKS,)),
            pltpu.SemaphoreType.DMA((_NUM_CHUNKS,)),
        ],
    )(inputs)


# speedup vs baseline: 1.0272x; 1.0272x over previous
import jax
import jax.numpy as jnp
from jax.experimental import pallas as pl
from jax.experimental.pallas import tpu as pltpu

_NUM_CHUNKS = 8


def _copy_kernel(in_hbm, out_hbm, buf, in_sems, out_sems):
    rows, _ = in_hbm.shape
    chunk = rows // _NUM_CHUNKS

    def copy_in(i):
        return pltpu.make_async_copy(
            in_hbm.at[pl.ds(i * chunk, chunk)], buf.at[i], in_sems.at[i])

    def copy_out(i):
        return pltpu.make_async_copy(
            buf.at[i], out_hbm.at[pl.ds(i * chunk, chunk)], out_sems.at[i])

    for i in range(_NUM_CHUNKS):
        copy_in(i).start(priority=i % 2)
    for i in range(_NUM_CHUNKS):
        copy_in(i).wait()
        copy_out(i).start(priority=i % 2)
    for i in range(_NUM_CHUNKS):
        copy_out(i).wait()


def kernel(inputs, embedding_table):
    del embedding_table
    rows, cols = inputs.shape
    chunk = rows // _NUM_CHUNKS
    return pl.pallas_call(
        _copy_kernel,
        out_shape=jax.ShapeDtypeStruct(inputs.shape, inputs.dtype),
        in_specs=[pl.BlockSpec(memory_space=pl.ANY)],
        out_specs=pl.BlockSpec(memory_space=pl.ANY),
        scratch_shapes=[
            pltpu.VMEM((_NUM_CHUNKS, chunk, cols), inputs.dtype),
            pltpu.SemaphoreType.DMA((_NUM_CHUNKS,)),
            pltpu.SemaphoreType.DMA((_NUM_CHUNKS,)),
        ],
    )(inputs)
